# Initial kernel scaffold; baseline (speedup 1.0000x reference)
import jax
import jax.numpy as jnp
from jax.experimental import pallas as pl


def kernel(x, W, expert_bias):
    raise NotImplementedError("write your pallas kernel here")


# speedup vs baseline: 1.1767x; 1.1767x over previous
import jax
import jax.numpy as jnp
from jax.experimental import pallas as pl

_TOP_K = 8
_BLOCK_T = 512


def _gate_kernel(x_ref, w_ref, b_ref, wout_ref, iout_ref):
    x = x_ref[...]
    w = w_ref[...]
    logits = jax.lax.dot_general(
        x, w, (((1,), (1,)), ((), ())), preferred_element_type=jnp.float32
    )
    sig = jax.nn.sigmoid(logits)
    work = logits + b_ref[...]
    n_exp = work.shape[1]
    iota = jax.lax.broadcasted_iota(jnp.int32, work.shape, 1)
    neg_inf = jnp.float32(-jnp.inf)
    ws, idxs = [], []
    for _ in range(_TOP_K):
        m = jnp.max(work, axis=1, keepdims=True)
        idx = jnp.min(jnp.where(work == m, iota, n_exp), axis=1, keepdims=True)
        sel = iota == idx
        ws.append(jnp.sum(jnp.where(sel, sig, 0.0), axis=1, keepdims=True))
        idxs.append(idx)
        work = jnp.where(sel, neg_inf, work)
    wmat = jnp.concatenate(ws, axis=1)
    imat = jnp.concatenate(idxs, axis=1)
    wout_ref[...] = wmat / jnp.sum(wmat, axis=1, keepdims=True)
    iout_ref[...] = imat


def kernel(x, W, expert_bias):
    t, d = x.shape
    e = W.shape[0]
    bias2d = expert_bias.reshape(1, e)
    wout, iout = pl.pallas_call(
        _gate_kernel,
        grid=(t // _BLOCK_T,),
        in_specs=[
            pl.BlockSpec((_BLOCK_T, d), lambda i: (i, 0)),
            pl.BlockSpec((e, d), lambda i: (0, 0)),
            pl.BlockSpec((1, e), lambda i: (0, 0)),
        ],
        out_specs=[
            pl.BlockSpec((_BLOCK_T, _TOP_K), lambda i: (i, 0)),
            pl.BlockSpec((_BLOCK_T, _TOP_K), lambda i: (i, 0)),
        ],
        out_shape=[
            jax.ShapeDtypeStruct((t, _TOP_K), jnp.float32),
            jax.ShapeDtypeStruct((t, _TOP_K), jnp.int32),
        ],
    )(x, W, bias2d)
    return (wout, iout)


# speedup vs baseline: 1.8458x; 1.5686x over previous
import jax
import jax.numpy as jnp
from jax.experimental import pallas as pl

_TOP_K = 8
_BLOCK_T = 512


def _gate_kernel(x_ref, w_ref, b_ref, wout_ref, iout_ref):
    x = x_ref[...]
    w = w_ref[...]
    logits = jax.lax.dot_general(
        x, w, (((1,), (1,)), ((), ())), preferred_element_type=jnp.float32
    )
    sig = jax.nn.sigmoid(logits)
    work = logits + b_ref[...]
    n_exp = work.shape[1]
    iota_f = jax.lax.broadcasted_iota(jnp.int32, work.shape, 1).astype(jnp.float32)
    key = iota_f + 0.5 * sig
    neg_inf = jnp.float32(-jnp.inf)
    big = jnp.float32(n_exp)
    vs = []
    for _ in range(_TOP_K):
        m = jnp.max(work, axis=1, keepdims=True)
        v = jnp.min(jnp.where(work == m, key, big), axis=1, keepdims=True)
        vs.append(v)
        work = jnp.where(key == v, neg_inf, work)
    vmat = jnp.concatenate(vs, axis=1)
    idx_f = jnp.floor(vmat)
    wmat = 2.0 * (vmat - idx_f)
    wout_ref[...] = wmat / jnp.sum(wmat, axis=1, keepdims=True)
    iout_ref[...] = idx_f.astype(jnp.int32)


def kernel(x, W, expert_bias):
    t, d = x.shape
    e = W.shape[0]
    bias2d = expert_bias.reshape(1, e)
    wout, iout = pl.pallas_call(
        _gate_kernel,
        grid=(t // _BLOCK_T,),
        in_specs=[
            pl.BlockSpec((_BLOCK_T, d), lambda i: (i, 0)),
            pl.BlockSpec((e, d), lambda i: (0, 0)),
            pl.BlockSpec((1, e), lambda i: (0, 0)),
        ],
        out_specs=[
            pl.BlockSpec((_BLOCK_T, _TOP_K), lambda i: (i, 0)),
            pl.BlockSpec((_BLOCK_T, _TOP_K), lambda i: (i, 0)),
        ],
        out_shape=[
            jax.ShapeDtypeStruct((t, _TOP_K), jnp.float32),
            jax.ShapeDtypeStruct((t, _TOP_K), jnp.int32),
        ],
    )(x, W, bias2d)
    return (wout, iout)


# speedup vs baseline: 3.2608x; 1.7666x over previous
import jax
import jax.numpy as jnp
from jax.experimental import pallas as pl

_TOP_K = 8
_BLOCK_T = 512


def _gate_kernel(x_ref, w_ref, b_ref, wout_ref, iout_ref):
    x = x_ref[...]
    w = w_ref[...]
    logits = jax.lax.dot_general(
        w, x, (((1,), (1,)), ((), ())), preferred_element_type=jnp.float32
    )
    sig = jax.nn.sigmoid(logits)
    work = logits + b_ref[...]
    n_exp = work.shape[0]
    iota_f = jax.lax.broadcasted_iota(jnp.int32, work.shape, 0).astype(jnp.float32)
    key = iota_f + 0.5 * sig
    neg_inf = jnp.float32(-jnp.inf)
    big = jnp.float32(n_exp)
    vs = []
    for _ in range(_TOP_K):
        m = jnp.max(work, axis=0, keepdims=True)
        v = jnp.min(jnp.where(work == m, key, big), axis=0, keepdims=True)
        vs.append(v)
        work = jnp.where(key == v, neg_inf, work)
    vmat = jnp.concatenate(vs, axis=0)
    idx_f = jnp.floor(vmat)
    wmat = 2.0 * (vmat - idx_f)
    wout_ref[...] = wmat / jnp.sum(wmat, axis=0, keepdims=True)
    iout_ref[...] = idx_f.astype(jnp.int32)


def kernel(x, W, expert_bias):
    t, d = x.shape
    e = W.shape[0]
    bias2d = expert_bias.reshape(e, 1)
    wout_t, iout_t = pl.pallas_call(
        _gate_kernel,
        grid=(t // _BLOCK_T,),
        in_specs=[
            pl.BlockSpec((_BLOCK_T, d), lambda i: (i, 0)),
            pl.BlockSpec((e, d), lambda i: (0, 0)),
            pl.BlockSpec((e, 1), lambda i: (0, 0)),
        ],
        out_specs=[
            pl.BlockSpec((_TOP_K, _BLOCK_T), lambda i: (0, i)),
            pl.BlockSpec((_TOP_K, _BLOCK_T), lambda i: (0, i)),
        ],
        out_shape=[
            jax.ShapeDtypeStruct((_TOP_K, t), jnp.float32),
            jax.ShapeDtypeStruct((_TOP_K, t), jnp.int32),
        ],
    )(x, W, bias2d)
    return (wout_t.T, iout_t.T)


# speedup vs baseline: 5.5407x; 1.6992x over previous
import jax
import jax.numpy as jnp
from jax.experimental import pallas as pl

_TOP_K = 8
_BLOCK_T = 2048


def _gate_kernel(x_ref, w_ref, b_ref, wout_ref, iout_ref):
    x = x_ref[...]
    w = w_ref[...]
    logits = jax.lax.dot_general(
        w, x, (((1,), (1,)), ((), ())), preferred_element_type=jnp.float32
    )
    sig = jax.nn.sigmoid(logits)
    work = logits + b_ref[...]
    n_exp = work.shape[0]
    iota_f = jax.lax.broadcasted_iota(jnp.int32, work.shape, 0).astype(jnp.float32)
    key = iota_f + 0.5 * sig
    neg_inf = jnp.float32(-jnp.inf)
    big = jnp.float32(n_exp)
    vs = []
    for _ in range(_TOP_K):
        m = jnp.max(work, axis=0, keepdims=True)
        v = jnp.min(jnp.where(work == m, key, big), axis=0, keepdims=True)
        vs.append(v)
        work = jnp.where(key == v, neg_inf, work)
    vmat = jnp.concatenate(vs, axis=0)
    idx_f = jnp.floor(vmat)
    wmat = 2.0 * (vmat - idx_f)
    wout_ref[...] = wmat / jnp.sum(wmat, axis=0, keepdims=True)
    iout_ref[...] = idx_f.astype(jnp.int32)


def kernel(x, W, expert_bias):
    t, d = x.shape
    e = W.shape[0]
    bias2d = expert_bias.reshape(e, 1)
    wout_t, iout_t = pl.pallas_call(
        _gate_kernel,
        grid=(t // _BLOCK_T,),
        in_specs=[
            pl.BlockSpec((_BLOCK_T, d), lambda i: (i, 0)),
            pl.BlockSpec((e, d), lambda i: (0, 0)),
            pl.BlockSpec((e, 1), lambda i: (0, 0)),
        ],
        out_specs=[
            pl.BlockSpec((_TOP_K, _BLOCK_T), lambda i: (0, i)),
            pl.BlockSpec((_TOP_K, _BLOCK_T), lambda i: (0, i)),
        ],
        out_shape=[
            jax.ShapeDtypeStruct((_TOP_K, t), jnp.float32),
            jax.ShapeDtypeStruct((_TOP_K, t), jnp.int32),
        ],
    )(x, W, bias2d)
    return (wout_t.T, iout_t.T)


# speedup vs baseline: 6.1326x; 1.1068x over previous
import jax
import jax.numpy as jnp
from jax.experimental import pallas as pl

_TOP_K = 8
_BLOCK_T = 4096


def _gate_kernel(x_ref, w_ref, b_ref, wout_ref, iout_ref):
    x = x_ref[...]
    w = w_ref[...]
    logits = jax.lax.dot_general(
        w, x, (((1,), (1,)), ((), ())), preferred_element_type=jnp.float32
    )
    sig = jax.nn.sigmoid(logits)
    work = logits + b_ref[...]
    n_exp = work.shape[0]
    iota_f = jax.lax.broadcasted_iota(jnp.int32, work.shape, 0).astype(jnp.float32)
    key = iota_f + 0.5 * sig
    neg_inf = jnp.float32(-jnp.inf)
    big = jnp.float32(n_exp)
    vs = []
    for _ in range(_TOP_K):
        m = jnp.max(work, axis=0, keepdims=True)
        v = jnp.min(jnp.where(work == m, key, big), axis=0, keepdims=True)
        vs.append(v)
        work = jnp.where(key == v, neg_inf, work)
    vmat = jnp.concatenate(vs, axis=0)
    idx_f = jnp.floor(vmat)
    wmat = 2.0 * (vmat - idx_f)
    wout_ref[...] = wmat / jnp.sum(wmat, axis=0, keepdims=True)
    iout_ref[...] = idx_f.astype(jnp.int32)


def kernel(x, W, expert_bias):
    t, d = x.shape
    e = W.shape[0]
    bias2d = expert_bias.reshape(e, 1)
    wout_t, iout_t = pl.pallas_call(
        _gate_kernel,
        grid=(t // _BLOCK_T,),
        in_specs=[
            pl.BlockSpec((_BLOCK_T, d), lambda i: (i, 0)),
            pl.BlockSpec((e, d), lambda i: (0, 0)),
            pl.BlockSpec((e, 1), lambda i: (0, 0)),
        ],
        out_specs=[
            pl.BlockSpec((_TOP_K, _BLOCK_T), lambda i: (0, i)),
            pl.BlockSpec((_TOP_K, _BLOCK_T), lambda i: (0, i)),
        ],
        out_shape=[
            jax.ShapeDtypeStruct((_TOP_K, t), jnp.float32),
            jax.ShapeDtypeStruct((_TOP_K, t), jnp.int32),
        ],
    )(x, W, bias2d)
    return (wout_t.T, iout_t.T)
